# baseline (speedup 1.0000x reference)
import functools

import jax
import jax.numpy as jnp
from jax import lax
from jax.experimental import pallas as pl
from jax.experimental.pallas import tpu as pltpu
from jax.experimental.pallas import tpu_sc as plsc

N_NODES = 10000
N_EDGES = 160000
F = 16

NC = 2
NS = 16
NW = NC * NS
EPW = N_EDGES // NW
EPT = N_EDGES // NS
CH = 125
NCH = EPW // CH
NCHG = EPT // CH
ROWS_PER_TILE = N_NODES // NS


@functools.cache
def _sc_gather_fn():
    mesh = plsc.VectorSubcoreMesh(core_axis_name="c", subcore_axis_name="s")

    @functools.partial(
        pl.kernel,
        mesh=mesh,
        out_type=jax.ShapeDtypeStruct((N_EDGES, F), jnp.float32),
        scratch_types=[
            pltpu.VMEM((NCH, CH), jnp.int32),
            pltpu.VMEM((EPW, F), jnp.float32),
            pltpu.VMEM_SHARED((N_NODES, F), jnp.float32),
            pltpu.SemaphoreType.DMA,
        ],
        compiler_params=pltpu.CompilerParams(use_tc_tiling_on_sc=False),
    )
    def _sc_gather(h_hbm, src_hbm, out_hbm, idx_v, rows_v, _spmem_probe, sem):
        wid = lax.axis_index("s") * NC + lax.axis_index("c")
        base = wid * EPW
        pltpu.sync_copy(src_hbm.at[pl.ds(wid * NCH, NCH)], idx_v)

        def fire(j, carry):
            pltpu.async_copy(
                h_hbm.at[idx_v.at[j]], rows_v.at[pl.ds(j * CH, CH)], sem
            )
            return carry

        lax.fori_loop(0, NCH, fire, 0)

        def drain(j, carry):
            pltpu.make_async_copy(
                h_hbm.at[idx_v.at[j]], rows_v.at[pl.ds(j * CH, CH)], sem
            ).wait()
            return carry

        lax.fori_loop(0, NCH, drain, 0)
        pltpu.sync_copy(rows_v, out_hbm.at[pl.ds(base, EPW)])

    return _sc_gather


@functools.cache
def _sc_layer_fn(with_gather):
    mesh = plsc.VectorSubcoreMesh(core_axis_name="c", subcore_axis_name="s")
    if with_gather:
        outs = (jax.ShapeDtypeStruct((NC, N_NODES, F), jnp.float32),
                jax.ShapeDtypeStruct((NC, N_EDGES, F), jnp.float32))
    else:
        outs = jax.ShapeDtypeStruct((NC, N_NODES, F), jnp.float32)

    @functools.partial(
        pl.kernel,
        mesh=mesh,
        out_type=outs,
        scratch_types=[
            pltpu.VMEM((NCHG, CH), jnp.int32),
            pltpu.VMEM((EPW, F), jnp.float32),
            pltpu.VMEM_SHARED((N_NODES, F), jnp.float32),
            pltpu.SemaphoreType.DMA,
        ],
        compiler_params=pltpu.CompilerParams(use_tc_tiling_on_sc=False),
    )
    def _sc_layer(msg_hbm, dst_hbm, src_hbm, pre_hbm, zeros_hbm,
                  aout_hbm, *rest):
        if with_gather:
            xsout_hbm, idx_v, rows_v, acc, sem = rest
        else:
            idx_v, rows_v, acc, sem = rest
        c = lax.axis_index("c")
        s = lax.axis_index("s")
        wid = s * NC + c
        nslice = pl.ds(s * ROWS_PER_TILE, ROWS_PER_TILE)

        @pl.when(c == 0)
        def _():
            pltpu.sync_copy(pre_hbm.at[nslice], acc.at[nslice])

        @pl.when(c == 1)
        def _():
            pltpu.sync_copy(zeros_hbm.at[nslice], acc.at[nslice])

        pltpu.sync_copy(dst_hbm.at[pl.ds(wid * NCH, NCH)],
                        idx_v.at[pl.ds(0, NCH)])
        pltpu.sync_copy(msg_hbm.at[pl.ds(wid * EPW, EPW)], rows_v)
        plsc.subcore_barrier()

        def chunk(j, carry):
            pltpu.sync_copy(
                rows_v.at[pl.ds(j * CH, CH)], acc.at[idx_v.at[j]], add=True
            )
            return carry

        lax.fori_loop(0, NCH, chunk, 0)
        plsc.subcore_barrier()
        pltpu.sync_copy(acc.at[nslice], aout_hbm.at[c, nslice])
        plsc.subcore_barrier()

        if with_gather:
            pltpu.sync_copy(src_hbm.at[pl.ds(s * NCHG, NCHG)], idx_v)
            table = aout_hbm.at[c]
            for half in range(2):
                jbase = half * NCH

                def fire(j, carry):
                    pltpu.async_copy(
                        table.at[idx_v.at[jbase + j]],
                        rows_v.at[pl.ds(j * CH, CH)], sem,
                    )
                    return carry

                lax.fori_loop(0, NCH, fire, 0)

                def drain(j, carry):
                    pltpu.make_async_copy(
                        table.at[idx_v.at[jbase + j]],
                        rows_v.at[pl.ds(j * CH, CH)], sem,
                    ).wait()
                    return carry

                lax.fori_loop(0, NCH, drain, 0)
                pltpu.sync_copy(
                    rows_v,
                    xsout_hbm.at[c, pl.ds(s * EPT + half * EPW, EPW)])

    return _sc_layer


BE = 3200
BNODE = N_NODES // (N_EDGES // BE)


def _msg_first_body(e_ref, xs_ref, h_ref, rep_ref, red_ref, root_ref, rb_ref,
                    w0, b0, w1, b1, w2, b2, w3, b3, msg_ref, pre_ref):
    _msg_common(e_ref[...], xs_ref[...], h_ref[...], rep_ref, red_ref,
                root_ref, rb_ref, w0, b0, w1, b1, w2, b2, w3, b3,
                msg_ref, pre_ref)


def _msg_mid_body(e_ref, xsp_ref, ap_ref, rep_ref, red_ref, root_ref, rb_ref,
                  w0, b0, w1, b1, w2, b2, w3, b3, msg_ref, pre_ref):
    xsp = xsp_ref[...]
    xs = jnp.maximum(xsp[0] + xsp[1], 0.0)
    ap = ap_ref[...]
    h = jnp.maximum(ap[0] + ap[1], 0.0)
    _msg_common(e_ref[...], xs, h, rep_ref, red_ref, root_ref, rb_ref,
                w0, b0, w1, b1, w2, b2, w3, b3, msg_ref, pre_ref)


def _msg_common(e, xs, h, rep_ref, red_ref, root_ref, rb_ref,
                w0, b0, w1, b1, w2, b2, w3, b3, msg_ref, pre_ref):
    t = jnp.maximum(
        jnp.dot(e, w0[...], preferred_element_type=jnp.float32) + b0[...], 0.0)
    t = jnp.maximum(
        jnp.dot(t, w1[...], preferred_element_type=jnp.float32) + b1[...], 0.0)
    t = jnp.maximum(
        jnp.dot(t, w2[...], preferred_element_type=jnp.float32) + b2[...], 0.0)
    k = jnp.dot(t, w3[...], preferred_element_type=jnp.float32) + b3[...]
    xr = jnp.dot(xs, rep_ref[...], preferred_element_type=jnp.float32)
    p = xr * k
    p = p[:, :128] + p[:, 128:]
    msg_ref[...] = jnp.dot(p, red_ref[...], preferred_element_type=jnp.float32)
    pre_ref[...] = (
        jnp.dot(h, root_ref[...], preferred_element_type=jnp.float32)
        + rb_ref[...])


def _msg_call(e, xs_or_xsp, h_or_ap, p, first):
    w0, w1, w2, w3 = p["kw"]
    b0, b1, b2, b3 = [b.reshape(1, -1) for b in p["kb"]]
    root = p["root"]
    rb = p["bias"].reshape(1, F)
    rep = jnp.kron(jnp.eye(F, dtype=jnp.float32),
                   jnp.ones((1, F), jnp.float32))
    red = jnp.kron(jnp.ones((8, 1), jnp.float32),
                   jnp.eye(F, dtype=jnp.float32))
    grid = (N_EDGES // BE,)
    edge_spec = pl.BlockSpec((BE, F), lambda i: (i, 0))

    def wspec(a):
        return pl.BlockSpec(a.shape, lambda i: (0, 0))

    if first:
        body = _msg_first_body
        xspec = edge_spec
        hspec = pl.BlockSpec((BNODE, F), lambda i: (i, 0))
    else:
        body = _msg_mid_body
        xspec = pl.BlockSpec((NC, BE, F), lambda i: (0, i, 0))
        hspec = pl.BlockSpec((NC, BNODE, F), lambda i: (0, i, 0))

    return pl.pallas_call(
        body,
        grid=grid,
        in_specs=[edge_spec, xspec, hspec, wspec(rep), wspec(red),
                  wspec(root), wspec(rb),
                  wspec(w0), wspec(b0), wspec(w1), wspec(b1),
                  wspec(w2), wspec(b2), wspec(w3), wspec(b3)],
        out_specs=[pl.BlockSpec((BE, F), lambda i: (i, 0)),
                   pl.BlockSpec((BNODE, F), lambda i: (i, 0))],
        out_shape=[jax.ShapeDtypeStruct((N_EDGES, F), jnp.float32),
                   jax.ShapeDtypeStruct((N_NODES, F), jnp.float32)],
    )(e, xs_or_xsp, h_or_ap, rep, red, root, rb,
      w0, b0, w1, b1, w2, b2, w3, b3)


def _head_body(ap_ref, w1, b1, w2, b2, w3, b3, out_ref):
    ap = ap_ref[...]
    h = jnp.maximum(ap[0] + ap[1], 0.0)
    t = jnp.sum(h, axis=0, keepdims=True)
    t = jnp.maximum(
        jnp.dot(t, w1[...], preferred_element_type=jnp.float32) + b1[...], 0.0)
    t = jnp.maximum(
        jnp.dot(t, w2[...], preferred_element_type=jnp.float32) + b2[...], 0.0)
    out_ref[...] = jnp.dot(t, w3[...], preferred_element_type=jnp.float32) + b3[...]


def _head_call(ap, mlp):
    w1, w2, w3 = [l["w"] for l in mlp]
    b1, b2, b3 = [l["b"].reshape(1, -1) for l in mlp]

    def fspec(a):
        return pl.BlockSpec(a.shape, lambda: tuple(0 for _ in a.shape))

    return pl.pallas_call(
        _head_body,
        in_specs=[fspec(ap), fspec(w1), fspec(b1), fspec(w2), fspec(b2),
                  fspec(w3), fspec(b3)],
        out_specs=pl.BlockSpec((1, 1), lambda: (0, 0)),
        out_shape=jax.ShapeDtypeStruct((1, 1), jnp.float32),
    )(ap, w1, b1, w2, b2, w3, b3)


def kernel(x, e, params, edge_index):
    src2 = edge_index[0].reshape(NW * NCH, CH)
    dst2 = edge_index[1].reshape(NW * NCH, CH)
    zeros = jnp.zeros((N_NODES, F), jnp.float32)
    ecc = params["ecc"]

    xs0 = _sc_gather_fn()(x, src2)
    msg, pre = _msg_call(e, xs0, x, ecc[0], first=True)
    for l in range(1, len(ecc)):
        ap, xsp = _sc_layer_fn(True)(msg, dst2, src2, pre, zeros)
        msg, pre = _msg_call(e, xsp, ap, ecc[l], first=False)
    ap = _sc_layer_fn(False)(msg, dst2, src2, pre, zeros)
    return _head_call(ap, params["mlp"])


# speedup vs baseline: 1.0004x; 1.0004x over previous
import functools

import jax
import jax.numpy as jnp
from jax import lax
from jax.experimental import pallas as pl
from jax.experimental.pallas import tpu as pltpu
from jax.experimental.pallas import tpu_sc as plsc

N_NODES = 10000
N_EDGES = 160000
F = 16

NC = 2
NS = 16
NW = NC * NS
EPW = N_EDGES // NW
EPT = N_EDGES // NS
CH = 125
NCH = EPW // CH
NCHG = EPT // CH
ROWS_PER_TILE = N_NODES // NS


@functools.cache
def _sc_gather_fn():
    mesh = plsc.VectorSubcoreMesh(core_axis_name="c", subcore_axis_name="s")

    @functools.partial(
        pl.kernel,
        mesh=mesh,
        out_type=jax.ShapeDtypeStruct((N_EDGES, F), jnp.float32),
        scratch_types=[
            pltpu.VMEM((NCH, CH), jnp.int32),
            pltpu.VMEM((EPW, F), jnp.float32),
            pltpu.VMEM_SHARED((N_NODES, F), jnp.float32),
            pltpu.SemaphoreType.DMA,
        ],
        compiler_params=pltpu.CompilerParams(use_tc_tiling_on_sc=False),
    )
    def _sc_gather(h_hbm, src_hbm, out_hbm, idx_v, rows_v, _spmem_probe, sem):
        wid = lax.axis_index("s") * NC + lax.axis_index("c")
        base = wid * EPW
        pltpu.sync_copy(src_hbm.at[pl.ds(wid * NCH, NCH)], idx_v)

        def fire(j, carry):
            pltpu.async_copy(
                h_hbm.at[idx_v.at[j]], rows_v.at[pl.ds(j * CH, CH)], sem
            )
            return carry

        lax.fori_loop(0, NCH, fire, 0)

        def drain(j, carry):
            pltpu.make_async_copy(
                h_hbm.at[idx_v.at[j]], rows_v.at[pl.ds(j * CH, CH)], sem
            ).wait()
            return carry

        lax.fori_loop(0, NCH, drain, 0)
        plsc.subcore_barrier()
        plsc.subcore_barrier()
        pltpu.sync_copy(rows_v, out_hbm.at[pl.ds(base, EPW)])

    return _sc_gather


@functools.cache
def _sc_layer_fn(with_gather):
    mesh = plsc.VectorSubcoreMesh(core_axis_name="c", subcore_axis_name="s")
    if with_gather:
        outs = (jax.ShapeDtypeStruct((NC, N_NODES, F), jnp.float32),
                jax.ShapeDtypeStruct((NC, N_EDGES, F), jnp.float32))
    else:
        outs = jax.ShapeDtypeStruct((NC, N_NODES, F), jnp.float32)

    @functools.partial(
        pl.kernel,
        mesh=mesh,
        out_type=outs,
        scratch_types=[
            pltpu.VMEM((NCHG, CH), jnp.int32),
            pltpu.VMEM((EPW, F), jnp.float32),
            pltpu.VMEM_SHARED((N_NODES, F), jnp.float32),
            pltpu.SemaphoreType.DMA,
        ],
        compiler_params=pltpu.CompilerParams(use_tc_tiling_on_sc=False),
    )
    def _sc_layer(msg_hbm, dst_hbm, src_hbm, pre_hbm, zeros_hbm,
                  aout_hbm, *rest):
        if with_gather:
            xsout_hbm, idx_v, rows_v, acc, sem = rest
        else:
            idx_v, rows_v, acc, sem = rest
        c = lax.axis_index("c")
        s = lax.axis_index("s")
        wid = s * NC + c
        nslice = pl.ds(s * ROWS_PER_TILE, ROWS_PER_TILE)

        @pl.when(c == 0)
        def _():
            pltpu.sync_copy(pre_hbm.at[nslice], acc.at[nslice])

        @pl.when(c == 1)
        def _():
            pltpu.sync_copy(zeros_hbm.at[nslice], acc.at[nslice])

        pltpu.sync_copy(dst_hbm.at[pl.ds(wid * NCH, NCH)],
                        idx_v.at[pl.ds(0, NCH)])
        pltpu.sync_copy(msg_hbm.at[pl.ds(wid * EPW, EPW)], rows_v)
        plsc.subcore_barrier()

        def chunk(j, carry):
            pltpu.sync_copy(
                rows_v.at[pl.ds(j * CH, CH)], acc.at[idx_v.at[j]], add=True
            )
            return carry

        lax.fori_loop(0, NCH, chunk, 0)
        plsc.subcore_barrier()
        pltpu.sync_copy(acc.at[nslice], aout_hbm.at[c, nslice])
        plsc.subcore_barrier()

        if with_gather:
            pltpu.sync_copy(src_hbm.at[pl.ds(s * NCHG, NCHG)], idx_v)
            table = aout_hbm.at[c]
            for half in range(2):
                jbase = half * NCH

                def fire(j, carry):
                    pltpu.async_copy(
                        table.at[idx_v.at[jbase + j]],
                        rows_v.at[pl.ds(j * CH, CH)], sem,
                    )
                    return carry

                lax.fori_loop(0, NCH, fire, 0)

                def drain(j, carry):
                    pltpu.make_async_copy(
                        table.at[idx_v.at[jbase + j]],
                        rows_v.at[pl.ds(j * CH, CH)], sem,
                    ).wait()
                    return carry

                lax.fori_loop(0, NCH, drain, 0)
                pltpu.sync_copy(
                    rows_v,
                    xsout_hbm.at[c, pl.ds(s * EPT + half * EPW, EPW)])

    return _sc_layer


BE = 3200
BNODE = N_NODES // (N_EDGES // BE)


def _msg_first_body(e_ref, xs_ref, h_ref, rep_ref, red_ref, root_ref, rb_ref,
                    w0, b0, w1, b1, w2, b2, w3, b3, msg_ref, pre_ref):
    _msg_common(e_ref[...], xs_ref[...], h_ref[...], rep_ref, red_ref,
                root_ref, rb_ref, w0, b0, w1, b1, w2, b2, w3, b3,
                msg_ref, pre_ref)


def _msg_mid_body(e_ref, xsp_ref, ap_ref, rep_ref, red_ref, root_ref, rb_ref,
                  w0, b0, w1, b1, w2, b2, w3, b3, msg_ref, pre_ref):
    xsp = xsp_ref[...]
    xs = jnp.maximum(xsp[0] + xsp[1], 0.0)
    ap = ap_ref[...]
    h = jnp.maximum(ap[0] + ap[1], 0.0)
    _msg_common(e_ref[...], xs, h, rep_ref, red_ref, root_ref, rb_ref,
                w0, b0, w1, b1, w2, b2, w3, b3, msg_ref, pre_ref)


def _msg_common(e, xs, h, rep_ref, red_ref, root_ref, rb_ref,
                w0, b0, w1, b1, w2, b2, w3, b3, msg_ref, pre_ref):
    t = jnp.maximum(
        jnp.dot(e, w0[...], preferred_element_type=jnp.float32) + b0[...], 0.0)
    t = jnp.maximum(
        jnp.dot(t, w1[...], preferred_element_type=jnp.float32) + b1[...], 0.0)
    t = jnp.maximum(
        jnp.dot(t, w2[...], preferred_element_type=jnp.float32) + b2[...], 0.0)
    k = jnp.dot(t, w3[...], preferred_element_type=jnp.float32) + b3[...]
    xr = jnp.dot(xs, rep_ref[...], preferred_element_type=jnp.float32)
    p = xr * k
    p = p[:, :128] + p[:, 128:]
    msg_ref[...] = jnp.dot(p, red_ref[...], preferred_element_type=jnp.float32)
    pre_ref[...] = (
        jnp.dot(h, root_ref[...], preferred_element_type=jnp.float32)
        + rb_ref[...])


def _msg_call(e, xs_or_xsp, h_or_ap, p, first):
    w0, w1, w2, w3 = p["kw"]
    b0, b1, b2, b3 = [b.reshape(1, -1) for b in p["kb"]]
    root = p["root"]
    rb = p["bias"].reshape(1, F)
    rep = jnp.kron(jnp.eye(F, dtype=jnp.float32),
                   jnp.ones((1, F), jnp.float32))
    red = jnp.kron(jnp.ones((8, 1), jnp.float32),
                   jnp.eye(F, dtype=jnp.float32))
    grid = (N_EDGES // BE,)
    edge_spec = pl.BlockSpec((BE, F), lambda i: (i, 0))

    def wspec(a):
        return pl.BlockSpec(a.shape, lambda i: (0, 0))

    if first:
        body = _msg_first_body
        xspec = edge_spec
        hspec = pl.BlockSpec((BNODE, F), lambda i: (i, 0))
    else:
        body = _msg_mid_body
        xspec = pl.BlockSpec((NC, BE, F), lambda i: (0, i, 0))
        hspec = pl.BlockSpec((NC, BNODE, F), lambda i: (0, i, 0))

    return pl.pallas_call(
        body,
        grid=grid,
        in_specs=[edge_spec, xspec, hspec, wspec(rep), wspec(red),
                  wspec(root), wspec(rb),
                  wspec(w0), wspec(b0), wspec(w1), wspec(b1),
                  wspec(w2), wspec(b2), wspec(w3), wspec(b3)],
        out_specs=[pl.BlockSpec((BE, F), lambda i: (i, 0)),
                   pl.BlockSpec((BNODE, F), lambda i: (i, 0))],
        out_shape=[jax.ShapeDtypeStruct((N_EDGES, F), jnp.float32),
                   jax.ShapeDtypeStruct((N_NODES, F), jnp.float32)],
    )(e, xs_or_xsp, h_or_ap, rep, red, root, rb,
      w0, b0, w1, b1, w2, b2, w3, b3)


def _head_body(ap_ref, w1, b1, w2, b2, w3, b3, out_ref):
    ap = ap_ref[...]
    h = jnp.maximum(ap[0] + ap[1], 0.0)
    t = jnp.sum(h, axis=0, keepdims=True)
    t = jnp.maximum(
        jnp.dot(t, w1[...], preferred_element_type=jnp.float32) + b1[...], 0.0)
    t = jnp.maximum(
        jnp.dot(t, w2[...], preferred_element_type=jnp.float32) + b2[...], 0.0)
    out_ref[...] = jnp.dot(t, w3[...], preferred_element_type=jnp.float32) + b3[...]


def _head_call(ap, mlp):
    w1, w2, w3 = [l["w"] for l in mlp]
    b1, b2, b3 = [l["b"].reshape(1, -1) for l in mlp]

    def fspec(a):
        return pl.BlockSpec(a.shape, lambda: tuple(0 for _ in a.shape))

    return pl.pallas_call(
        _head_body,
        in_specs=[fspec(ap), fspec(w1), fspec(b1), fspec(w2), fspec(b2),
                  fspec(w3), fspec(b3)],
        out_specs=pl.BlockSpec((1, 1), lambda: (0, 0)),
        out_shape=jax.ShapeDtypeStruct((1, 1), jnp.float32),
    )(ap, w1, b1, w2, b2, w3, b3)


def kernel(x, e, params, edge_index):
    src2 = edge_index[0].reshape(NW * NCH, CH)
    dst2 = edge_index[1].reshape(NW * NCH, CH)
    zeros = jnp.zeros((N_NODES, F), jnp.float32)
    ecc = params["ecc"]

    xs0 = _sc_gather_fn()(x, src2)
    msg, pre = _msg_call(e, xs0, x, ecc[0], first=True)
    for l in range(1, len(ecc)):
        ap, xsp = _sc_layer_fn(True)(msg, dst2, src2, pre, zeros)
        msg, pre = _msg_call(e, xsp, ap, ecc[l], first=False)
    ap = _sc_layer_fn(False)(msg, dst2, src2, pre, zeros)
    return _head_call(ap, params["mlp"])


# speedup vs baseline: 2.0000x; 1.9992x over previous
import functools

import jax
import jax.numpy as jnp
from jax import lax
from jax.experimental import pallas as pl
from jax.experimental.pallas import tpu as pltpu
from jax.experimental.pallas import tpu_sc as plsc

N_NODES = 10000
N_EDGES = 160000
F = 16

NC = 2
NS = 16
NW = NC * NS
EPW = N_EDGES // NW
EPT = N_EDGES // NS
CH = 125
NCH = EPW // CH
NCHG = EPT // CH
ROWS_PER_TILE = N_NODES // NS


@functools.cache
def _sc_gather_fn():
    mesh = plsc.VectorSubcoreMesh(core_axis_name="c", subcore_axis_name="s")

    @functools.partial(
        pl.kernel,
        mesh=mesh,
        out_type=jax.ShapeDtypeStruct((N_EDGES, F), jnp.float32),
        scratch_types=[
            pltpu.VMEM((NCH, CH), jnp.int32),
            pltpu.VMEM((EPW, F), jnp.float32),
            pltpu.SemaphoreType.DMA,
        ],
        compiler_params=pltpu.CompilerParams(use_tc_tiling_on_sc=False),
    )
    def _sc_gather(h_hbm, src_hbm, out_hbm, idx_v, rows_v, sem):
        wid = lax.axis_index("s") * NC + lax.axis_index("c")
        base = wid * EPW
        pltpu.sync_copy(src_hbm.at[pl.ds(wid * NCH, NCH)], idx_v)

        def fire(j, carry):
            pltpu.async_copy(
                h_hbm.at[idx_v.at[j]], rows_v.at[pl.ds(j * CH, CH)], sem
            )
            return carry

        lax.fori_loop(0, NCH, fire, 0)

        def drain(j, carry):
            pltpu.make_async_copy(
                h_hbm.at[idx_v.at[j]], rows_v.at[pl.ds(j * CH, CH)], sem
            ).wait()
            return carry

        lax.fori_loop(0, NCH, drain, 0)
        pltpu.sync_copy(rows_v, out_hbm.at[pl.ds(base, EPW)])

    return _sc_gather


@functools.cache
def _sc_layer_fn(with_gather):
    mesh = plsc.VectorSubcoreMesh(core_axis_name="c", subcore_axis_name="s")
    if with_gather:
        outs = (jax.ShapeDtypeStruct((NC, N_NODES, F), jnp.float32),
                jax.ShapeDtypeStruct((NC, N_EDGES, F), jnp.float32))
    else:
        outs = jax.ShapeDtypeStruct((NC, N_NODES, F), jnp.float32)

    @functools.partial(
        pl.kernel,
        mesh=mesh,
        out_type=outs,
        scratch_types=[
            pltpu.VMEM((NCHG, CH), jnp.int32),
            pltpu.VMEM((EPW, F), jnp.float32),
            pltpu.VMEM_SHARED((N_NODES, F), jnp.float32),
            pltpu.SemaphoreType.DMA,
        ],
        compiler_params=pltpu.CompilerParams(use_tc_tiling_on_sc=False),
    )
    def _sc_layer(msg_hbm, dst_hbm, src_hbm, pre_hbm, zeros_hbm,
                  aout_hbm, *rest):
        if with_gather:
            xsout_hbm, idx_v, rows_v, acc, sem = rest
        else:
            idx_v, rows_v, acc, sem = rest
        c = lax.axis_index("c")
        s = lax.axis_index("s")
        wid = s * NC + c
        nslice = pl.ds(s * ROWS_PER_TILE, ROWS_PER_TILE)

        @pl.when(c == 0)
        def _():
            pltpu.sync_copy(pre_hbm.at[nslice], acc.at[nslice])

        @pl.when(c == 1)
        def _():
            pltpu.sync_copy(zeros_hbm.at[nslice], acc.at[nslice])

        pltpu.sync_copy(dst_hbm.at[pl.ds(wid * NCH, NCH)],
                        idx_v.at[pl.ds(0, NCH)])
        pltpu.sync_copy(msg_hbm.at[pl.ds(wid * EPW, EPW)], rows_v)
        plsc.subcore_barrier()

        def chunk(j, carry):
            pltpu.sync_copy(
                rows_v.at[pl.ds(j * CH, CH)], acc.at[idx_v.at[j]], add=True
            )
            return carry

        lax.fori_loop(0, NCH, chunk, 0)
        plsc.subcore_barrier()
        pltpu.sync_copy(acc.at[nslice], aout_hbm.at[c, nslice])
        plsc.subcore_barrier()

        if with_gather:
            pltpu.sync_copy(src_hbm.at[pl.ds(s * NCHG, NCHG)], idx_v)
            table = aout_hbm.at[c]
            for half in range(2):
                jbase = half * NCH

                def fire(j, carry):
                    pltpu.async_copy(
                        table.at[idx_v.at[jbase + j]],
                        rows_v.at[pl.ds(j * CH, CH)], sem,
                    )
                    return carry

                lax.fori_loop(0, NCH, fire, 0)

                def drain(j, carry):
                    pltpu.make_async_copy(
                        table.at[idx_v.at[jbase + j]],
                        rows_v.at[pl.ds(j * CH, CH)], sem,
                    ).wait()
                    return carry

                lax.fori_loop(0, NCH, drain, 0)
                pltpu.sync_copy(
                    rows_v,
                    xsout_hbm.at[c, pl.ds(s * EPT + half * EPW, EPW)])

    return _sc_layer


BE = 3200
BEP = BE * F // 128
BNODE = N_NODES // (N_EDGES // BE)


def _msg_first_body(e_ref, xs_ref, h_ref, rep_ref, red_ref, root_ref, rb_ref,
                    w0, b0, w1, b1, w2, b2, w3, b3, msg_ref, pre_ref):
    _msg_common(e_ref[...], xs_ref[...], h_ref[...], rep_ref, red_ref,
                root_ref, rb_ref, w0, b0, w1, b1, w2, b2, w3, b3,
                msg_ref, pre_ref)


def _msg_mid_body(e_ref, xsp_ref, ap_ref, rep_ref, red_ref, root_ref, rb_ref,
                  w0, b0, w1, b1, w2, b2, w3, b3, msg_ref, pre_ref):
    xsp = xsp_ref[...]
    xs = jnp.maximum(xsp[0] + xsp[1], 0.0)
    ap = ap_ref[...]
    h = jnp.maximum(ap[0] + ap[1], 0.0)
    _msg_common(e_ref[...], xs, h, rep_ref, red_ref, root_ref, rb_ref,
                w0, b0, w1, b1, w2, b2, w3, b3, msg_ref, pre_ref)


def _msg_common(e_p, xs_p, h, rep_ref, red_ref, root_ref, rb_ref,
                w0, b0, w1, b1, w2, b2, w3, b3, msg_ref, pre_ref):
    t = jnp.maximum(
        jnp.dot(e_p, w0[...], preferred_element_type=jnp.float32) + b0[...], 0.0)
    t = jnp.maximum(
        jnp.dot(t, w1[...], preferred_element_type=jnp.float32) + b1[...], 0.0)
    t = jnp.maximum(
        jnp.dot(t, w2[...], preferred_element_type=jnp.float32) + b2[...], 0.0)
    k = jnp.dot(t, w3[...], preferred_element_type=jnp.float32) + b3[...]
    xr = jnp.dot(xs_p, rep_ref[...], preferred_element_type=jnp.float32)
    p = xr * k
    parts = [p[:, 256 * j:256 * j + 128] + p[:, 256 * j + 128:256 * (j + 1)]
             for j in range(8)]
    p2 = jnp.concatenate(parts, axis=1)
    msg_ref[...] = jnp.dot(p2, red_ref[...], preferred_element_type=jnp.float32)
    pre_ref[...] = (
        jnp.dot(h, root_ref[...], preferred_element_type=jnp.float32)
        + rb_ref[...])


def _msg_call(e, xs_or_xsp, h_or_ap, p, first):
    i8 = jnp.eye(8, dtype=jnp.float32)
    w0, w1, w2, w3 = [jnp.kron(i8, w) for w in p["kw"]]
    b0, b1, b2, b3 = [jnp.tile(b.reshape(1, -1), (1, 8)) for b in p["kb"]]
    root = p["root"]
    rb = p["bias"].reshape(1, F)
    rep = jnp.kron(i8, jnp.kron(jnp.eye(F, dtype=jnp.float32),
                                jnp.ones((1, F), jnp.float32)))
    red = jnp.kron(i8, jnp.kron(jnp.ones((8, 1), jnp.float32),
                                jnp.eye(F, dtype=jnp.float32)))
    grid = (N_EDGES // BE,)
    EP = N_EDGES * F // 128
    edge_spec = pl.BlockSpec((BEP, 128), lambda i: (i, 0))
    e_p = e.reshape(EP, 128)

    def wspec(a):
        return pl.BlockSpec(a.shape, lambda i: (0, 0))

    if first:
        body = _msg_first_body
        xs_in = xs_or_xsp.reshape(EP, 128)
        xspec = edge_spec
        hspec = pl.BlockSpec((BNODE, F), lambda i: (i, 0))
    else:
        body = _msg_mid_body
        xs_in = xs_or_xsp.reshape(NC, EP, 128)
        xspec = pl.BlockSpec((NC, BEP, 128), lambda i: (0, i, 0))
        hspec = pl.BlockSpec((NC, BNODE, F), lambda i: (0, i, 0))

    msg_p, pre = pl.pallas_call(
        body,
        grid=grid,
        in_specs=[edge_spec, xspec, hspec, wspec(rep), wspec(red),
                  wspec(root), wspec(rb),
                  wspec(w0), wspec(b0), wspec(w1), wspec(b1),
                  wspec(w2), wspec(b2), wspec(w3), wspec(b3)],
        out_specs=[pl.BlockSpec((BEP, 128), lambda i: (i, 0)),
                   pl.BlockSpec((BNODE, F), lambda i: (i, 0))],
        out_shape=[jax.ShapeDtypeStruct((EP, 128), jnp.float32),
                   jax.ShapeDtypeStruct((N_NODES, F), jnp.float32)],
    )(e_p, xs_in, h_or_ap, rep, red, root, rb,
      w0, b0, w1, b1, w2, b2, w3, b3)
    return msg_p.reshape(N_EDGES, F), pre


def _head_body(ap_ref, w1, b1, w2, b2, w3, b3, out_ref):
    ap = ap_ref[...]
    h = jnp.maximum(ap[0] + ap[1], 0.0)
    t = jnp.sum(h, axis=0, keepdims=True)
    t = jnp.maximum(
        jnp.dot(t, w1[...], preferred_element_type=jnp.float32) + b1[...], 0.0)
    t = jnp.maximum(
        jnp.dot(t, w2[...], preferred_element_type=jnp.float32) + b2[...], 0.0)
    out_ref[...] = jnp.dot(t, w3[...], preferred_element_type=jnp.float32) + b3[...]


def _head_call(ap, mlp):
    w1, w2, w3 = [l["w"] for l in mlp]
    b1, b2, b3 = [l["b"].reshape(1, -1) for l in mlp]

    def fspec(a):
        return pl.BlockSpec(a.shape, lambda: tuple(0 for _ in a.shape))

    return pl.pallas_call(
        _head_body,
        in_specs=[fspec(ap), fspec(w1), fspec(b1), fspec(w2), fspec(b2),
                  fspec(w3), fspec(b3)],
        out_specs=pl.BlockSpec((1, 1), lambda: (0, 0)),
        out_shape=jax.ShapeDtypeStruct((1, 1), jnp.float32),
    )(ap, w1, b1, w2, b2, w3, b3)


def kernel(x, e, params, edge_index):
    src2 = edge_index[0].reshape(NW * NCH, CH)
    dst2 = edge_index[1].reshape(NW * NCH, CH)
    zeros = jnp.zeros((N_NODES, F), jnp.float32)
    ecc = params["ecc"]

    xs0 = _sc_gather_fn()(x, src2)
    msg, pre = _msg_call(e, xs0, x, ecc[0], first=True)
    for l in range(1, len(ecc)):
        ap, xsp = _sc_layer_fn(True)(msg, dst2, src2, pre, zeros)
        msg, pre = _msg_call(e, xsp, ap, ecc[l], first=False)
    ap = _sc_layer_fn(False)(msg, dst2, src2, pre, zeros)
    return _head_call(ap, params["mlp"])


# speedup vs baseline: 2.1121x; 1.0560x over previous
import functools

import jax
import jax.numpy as jnp
from jax import lax
from jax.experimental import pallas as pl
from jax.experimental.pallas import tpu as pltpu
from jax.experimental.pallas import tpu_sc as plsc

N_NODES = 10000
N_EDGES = 160000
F = 16

NC = 2
NS = 16
NW = NC * NS
EPW = N_EDGES // NW
EPT = N_EDGES // NS
CH = 125
NCH = EPW // CH
NCHG = EPT // CH
ROWS_PER_TILE = N_NODES // NS


@functools.cache
def _sc_gather_fn():
    mesh = plsc.VectorSubcoreMesh(core_axis_name="c", subcore_axis_name="s")

    @functools.partial(
        pl.kernel,
        mesh=mesh,
        out_type=jax.ShapeDtypeStruct((N_EDGES, F), jnp.float32),
        scratch_types=[
            pltpu.VMEM((NCH, CH), jnp.int32),
            pltpu.VMEM((EPW, F), jnp.float32),
            pltpu.SemaphoreType.DMA,
        ],
        compiler_params=pltpu.CompilerParams(use_tc_tiling_on_sc=False),
    )
    def _sc_gather(h_hbm, src_hbm, out_hbm, idx_v, rows_v, sem):
        wid = lax.axis_index("s") * NC + lax.axis_index("c")
        base = wid * EPW
        pltpu.sync_copy(src_hbm.at[pl.ds(wid * NCH, NCH)], idx_v)

        def fire(j, carry):
            pltpu.async_copy(
                h_hbm.at[idx_v.at[j]], rows_v.at[pl.ds(j * CH, CH)], sem
            )
            return carry

        lax.fori_loop(0, NCH, fire, 0)

        def drain(j, carry):
            pltpu.make_async_copy(
                h_hbm.at[idx_v.at[j]], rows_v.at[pl.ds(j * CH, CH)], sem
            ).wait()
            return carry

        lax.fori_loop(0, NCH, drain, 0)
        pltpu.sync_copy(rows_v, out_hbm.at[pl.ds(base, EPW)])

    return _sc_gather


@functools.cache
def _sc_layer_fn(with_gather):
    mesh = plsc.VectorSubcoreMesh(core_axis_name="c", subcore_axis_name="s")
    if with_gather:
        outs = (jax.ShapeDtypeStruct((NC, N_NODES, F), jnp.float32),
                jax.ShapeDtypeStruct((NC, N_EDGES, F), jnp.float32))
    else:
        outs = jax.ShapeDtypeStruct((NC, N_NODES, F), jnp.float32)

    @functools.partial(
        pl.kernel,
        mesh=mesh,
        out_type=outs,
        scratch_types=[
            pltpu.VMEM((NCHG, CH), jnp.int32),
            pltpu.VMEM((EPW, F), jnp.float32),
            pltpu.VMEM_SHARED((N_NODES, F), jnp.float32),
            pltpu.SemaphoreType.DMA,
        ],
        compiler_params=pltpu.CompilerParams(use_tc_tiling_on_sc=False),
    )
    def _sc_layer(msg_hbm, dst_hbm, src_hbm, pre_hbm, zeros_hbm,
                  aout_hbm, *rest):
        if with_gather:
            xsout_hbm, idx_v, rows_v, acc, sem = rest
        else:
            idx_v, rows_v, acc, sem = rest
        c = lax.axis_index("c")
        s = lax.axis_index("s")
        wid = s * NC + c
        nslice = pl.ds(s * ROWS_PER_TILE, ROWS_PER_TILE)

        @pl.when(c == 0)
        def _():
            pltpu.sync_copy(pre_hbm.at[nslice], acc.at[nslice])

        @pl.when(c == 1)
        def _():
            pltpu.sync_copy(zeros_hbm.at[nslice], acc.at[nslice])

        pltpu.sync_copy(dst_hbm.at[pl.ds(wid * NCH, NCH)],
                        idx_v.at[pl.ds(0, NCH)])
        pltpu.sync_copy(msg_hbm.at[pl.ds(wid * EPW, EPW)], rows_v)
        plsc.subcore_barrier()

        def chunk(j, carry):
            pltpu.sync_copy(
                rows_v.at[pl.ds(j * CH, CH)], acc.at[idx_v.at[j]], add=True
            )
            return carry

        lax.fori_loop(0, NCH, chunk, 0)
        plsc.subcore_barrier()
        pltpu.sync_copy(acc.at[nslice], aout_hbm.at[c, nslice])
        plsc.subcore_barrier()

        if with_gather:
            pltpu.sync_copy(src_hbm.at[pl.ds(s * NCHG, NCHG)], idx_v)
            table = aout_hbm.at[c]
            for half in range(2):
                jbase = half * NCH

                def fire(j, carry):
                    pltpu.async_copy(
                        table.at[idx_v.at[jbase + j]],
                        rows_v.at[pl.ds(j * CH, CH)], sem,
                    )
                    return carry

                lax.fori_loop(0, NCH, fire, 0)

                def drain(j, carry):
                    pltpu.make_async_copy(
                        table.at[idx_v.at[jbase + j]],
                        rows_v.at[pl.ds(j * CH, CH)], sem,
                    ).wait()
                    return carry

                lax.fori_loop(0, NCH, drain, 0)
                pltpu.sync_copy(
                    rows_v,
                    xsout_hbm.at[c, pl.ds(s * EPT + half * EPW, EPW)])

    return _sc_layer


BE = 6400
BEP = BE * F // 128
BNODE = N_NODES // (N_EDGES // BE)


def _msg_first_body(e_ref, xs_ref, h_ref, rep_ref, red_ref, root_ref, rb_ref,
                    w0, b0, w1, b1, w2, b2, w3, b3, msg_ref, pre_ref):
    _msg_common(e_ref[...], xs_ref[...], h_ref[...], rep_ref, red_ref,
                root_ref, rb_ref, w0, b0, w1, b1, w2, b2, w3, b3,
                msg_ref, pre_ref)


def _msg_mid_body(e_ref, xsp_ref, ap_ref, rep_ref, red_ref, root_ref, rb_ref,
                  w0, b0, w1, b1, w2, b2, w3, b3, msg_ref, pre_ref):
    xsp = xsp_ref[...]
    xs = jnp.maximum(xsp[0] + xsp[1], 0.0)
    ap = ap_ref[...]
    h = jnp.maximum(ap[0] + ap[1], 0.0)
    _msg_common(e_ref[...], xs, h, rep_ref, red_ref, root_ref, rb_ref,
                w0, b0, w1, b1, w2, b2, w3, b3, msg_ref, pre_ref)


def _msg_common(e_p, xs_p, h, rep_ref, red_ref, root_ref, rb_ref,
                w0, b0, w1, b1, w2, b2, w3, b3, msg_ref, pre_ref):
    t = jnp.maximum(
        jnp.dot(e_p, w0[...], preferred_element_type=jnp.float32) + b0[...], 0.0)
    t = jnp.maximum(
        jnp.dot(t, w1[...], preferred_element_type=jnp.float32) + b1[...], 0.0)
    t = jnp.maximum(
        jnp.dot(t, w2[...], preferred_element_type=jnp.float32) + b2[...], 0.0)
    k = jnp.dot(t, w3[...], preferred_element_type=jnp.float32) + b3[...]
    xr = jnp.dot(xs_p, rep_ref[...], preferred_element_type=jnp.float32)
    p = xr * k
    parts = [p[:, 256 * j:256 * j + 128] + p[:, 256 * j + 128:256 * (j + 1)]
             for j in range(8)]
    p2 = jnp.concatenate(parts, axis=1)
    msg_ref[...] = jnp.dot(p2, red_ref[...], preferred_element_type=jnp.float32)
    pre_ref[...] = (
        jnp.dot(h, root_ref[...], preferred_element_type=jnp.float32)
        + rb_ref[...])


def _msg_call(e, xs_or_xsp, h_or_ap, p, first):
    i8 = jnp.eye(8, dtype=jnp.float32)
    w0, w1, w2, w3 = [jnp.kron(i8, w) for w in p["kw"]]
    b0, b1, b2, b3 = [jnp.tile(b.reshape(1, -1), (1, 8)) for b in p["kb"]]
    root = p["root"]
    rb = p["bias"].reshape(1, F)
    rep = jnp.kron(i8, jnp.kron(jnp.eye(F, dtype=jnp.float32),
                                jnp.ones((1, F), jnp.float32)))
    red = jnp.kron(i8, jnp.kron(jnp.ones((8, 1), jnp.float32),
                                jnp.eye(F, dtype=jnp.float32)))
    grid = (N_EDGES // BE,)
    EP = N_EDGES * F // 128
    edge_spec = pl.BlockSpec((BEP, 128), lambda i: (i, 0))
    e_p = e.reshape(EP, 128)

    def wspec(a):
        return pl.BlockSpec(a.shape, lambda i: (0, 0))

    if first:
        body = _msg_first_body
        xs_in = xs_or_xsp.reshape(EP, 128)
        xspec = edge_spec
        hspec = pl.BlockSpec((BNODE, F), lambda i: (i, 0))
    else:
        body = _msg_mid_body
        xs_in = xs_or_xsp.reshape(NC, EP, 128)
        xspec = pl.BlockSpec((NC, BEP, 128), lambda i: (0, i, 0))
        hspec = pl.BlockSpec((NC, BNODE, F), lambda i: (0, i, 0))

    msg_p, pre = pl.pallas_call(
        body,
        grid=grid,
        in_specs=[edge_spec, xspec, hspec, wspec(rep), wspec(red),
                  wspec(root), wspec(rb),
                  wspec(w0), wspec(b0), wspec(w1), wspec(b1),
                  wspec(w2), wspec(b2), wspec(w3), wspec(b3)],
        out_specs=[pl.BlockSpec((BEP, 128), lambda i: (i, 0)),
                   pl.BlockSpec((BNODE, F), lambda i: (i, 0))],
        out_shape=[jax.ShapeDtypeStruct((EP, 128), jnp.float32),
                   jax.ShapeDtypeStruct((N_NODES, F), jnp.float32)],
    )(e_p, xs_in, h_or_ap, rep, red, root, rb,
      w0, b0, w1, b1, w2, b2, w3, b3)
    return msg_p.reshape(N_EDGES, F), pre


def _head_body(ap_ref, w1, b1, w2, b2, w3, b3, out_ref):
    ap = ap_ref[...]
    h = jnp.maximum(ap[0] + ap[1], 0.0)
    t = jnp.sum(h, axis=0, keepdims=True)
    t = jnp.maximum(
        jnp.dot(t, w1[...], preferred_element_type=jnp.float32) + b1[...], 0.0)
    t = jnp.maximum(
        jnp.dot(t, w2[...], preferred_element_type=jnp.float32) + b2[...], 0.0)
    out_ref[...] = jnp.dot(t, w3[...], preferred_element_type=jnp.float32) + b3[...]


def _head_call(ap, mlp):
    w1, w2, w3 = [l["w"] for l in mlp]
    b1, b2, b3 = [l["b"].reshape(1, -1) for l in mlp]

    def fspec(a):
        return pl.BlockSpec(a.shape, lambda: tuple(0 for _ in a.shape))

    return pl.pallas_call(
        _head_body,
        in_specs=[fspec(ap), fspec(w1), fspec(b1), fspec(w2), fspec(b2),
                  fspec(w3), fspec(b3)],
        out_specs=pl.BlockSpec((1, 1), lambda: (0, 0)),
        out_shape=jax.ShapeDtypeStruct((1, 1), jnp.float32),
    )(ap, w1, b1, w2, b2, w3, b3)


def kernel(x, e, params, edge_index):
    src2 = edge_index[0].reshape(NW * NCH, CH)
    dst2 = edge_index[1].reshape(NW * NCH, CH)
    zeros = jnp.zeros((N_NODES, F), jnp.float32)
    ecc = params["ecc"]

    xs0 = _sc_gather_fn()(x, src2)
    msg, pre = _msg_call(e, xs0, x, ecc[0], first=True)
    for l in range(1, len(ecc)):
        ap, xsp = _sc_layer_fn(True)(msg, dst2, src2, pre, zeros)
        msg, pre = _msg_call(e, xsp, ap, ecc[l], first=False)
    ap = _sc_layer_fn(False)(msg, dst2, src2, pre, zeros)
    return _head_call(ap, params["mlp"])
